# baseline (speedup 1.0000x reference)
import jax
import jax.numpy as jnp
from jax.experimental import pallas as pl
from jax.experimental.pallas import tpu as pltpu

_E = 8
_K = 2
_D = 768
_C = 3
_S = 512
_F = _C * _S
_EPS = 1e-9


def _routing_body(logits_ref, mask_ref, x_ref, xb_ref, sb_ref, se_ref,
                  gf_ref, gl_ref):
    n_b = logits_ref.shape[0]
    n_l = x_ref.shape[1]
    n_a = n_b * _K

    lg = logits_ref[...]
    mk = mask_ref[...]
    m = jnp.where(mk == 1.0, 1.0, 0.0).astype(jnp.float32)
    z = lg - jnp.max(lg, axis=1, keepdims=True)
    ez = jnp.exp(z)
    probs = ez / jnp.sum(ez, axis=1, keepdims=True)
    pm = probs * m
    iota = jax.lax.broadcasted_iota(jnp.int32, pm.shape, 1)
    m1 = jnp.max(pm, axis=1, keepdims=True)
    a1 = jnp.min(jnp.where(pm == m1, iota, _E), axis=1, keepdims=True)
    pm2 = jnp.where(iota == a1, -1.0, pm)
    m2 = jnp.max(pm2, axis=1, keepdims=True)
    a2 = jnp.min(jnp.where(pm2 == m2, iota, _E), axis=1, keepdims=True)
    topk = jnp.logical_or(iota == a1, iota == a2)
    act = jnp.where(topk, 1.0, 0.0).astype(jnp.float32)
    gts = pm * act
    dn = jnp.sum(gts, axis=1, keepdims=True) + _EPS
    gates = gts / dn
    s = jnp.sum(pm) / jnp.float32(n_b)
    gl_ref[...] = ((1.0 - s) * (1.0 - s)).reshape(1, 1)

    ltri = (jax.lax.broadcasted_iota(jnp.int32, (n_b, n_b), 0)
            > jax.lax.broadcasted_iota(jnp.int32, (n_b, n_b), 1)
            ).astype(jnp.float32)
    pos = jnp.dot(ltri, act, preferred_element_type=jnp.float32)
    counts = jnp.sum(act, axis=0, keepdims=True)
    sut = (jax.lax.broadcasted_iota(jnp.int32, (_E, _E), 0)
           < jax.lax.broadcasted_iota(jnp.int32, (_E, _E), 1)
           ).astype(jnp.float32)
    offs = jnp.dot(counts, sut, preferred_element_type=jnp.float32)
    gpos = pos + offs

    act_t = act.T
    gpos_t = gpos.T
    gates_t = gates.T

    slot = jax.lax.broadcasted_iota(jnp.int32, (n_a, n_b), 0)
    bcol = jax.lax.broadcasted_iota(jnp.int32, (n_b, 1), 0).astype(jnp.float32)
    ones = jnp.ones((n_b, 1), jnp.float32)
    sb_acc = jnp.zeros((n_a, 1), jnp.float32)
    se_acc = jnp.zeros((n_a, 1), jnp.float32)
    gf_acc = jnp.zeros((n_a, 1), jnp.float32)
    for e in range(_E):
        arow = act_t[e:e + 1, :]
        grow = gpos_t[e:e + 1, :]
        gaterow = gates_t[e:e + 1, :]
        p = jnp.where(slot == grow.astype(jnp.int32), 1.0, 0.0) * arow
        sb_acc += jnp.dot(p, bcol, preferred_element_type=jnp.float32)
        se_acc += jnp.float32(e) * jnp.dot(p, ones,
                                           preferred_element_type=jnp.float32)
        gf_acc += jnp.dot(p * gaterow, ones,
                          preferred_element_type=jnp.float32)
    sb_ref[...] = sb_acc.astype(jnp.int32)
    se_ref[...] = se_acc.astype(jnp.int32)
    gf_ref[...] = gf_acc

    for c in range(_C):
        xb_ref[:, c * _S:(c + 1) * _S] = (
            x_ref[:, :, c, :].reshape(n_b * n_l, _S).astype(jnp.bfloat16))


def _dispatch_body(sb_ref, se_ref, gf_ref, x_ref, w_ref, b_ref,
                   out_ref, wb_ref):
    a = pl.program_id(0)
    n_l = x_ref.shape[1]

    @pl.when(a == 0)
    def _zero():
        out_ref[...] = jnp.zeros_like(out_ref)

    se_cur = se_ref[a, 0]
    se_prev = se_ref[jnp.maximum(a - 1, 0), 0]

    @pl.when(jnp.logical_or(a == 0, se_cur != se_prev))
    def _cast_w():
        wb_ref[...] = w_ref[0].astype(jnp.bfloat16)

    y = jnp.dot(x_ref[0], wb_ref[...], preferred_element_type=jnp.float32)
    g = gf_ref[a, 0]
    bias = b_ref[pl.ds(se_cur, 1), :]
    contrib = (g * (y + bias)).reshape(1, n_l, _D)
    sbv = sb_ref[a, 0]
    out_ref[pl.ds(sbv, 1)] += contrib


def kernel(cycle_curve_data, logits, moe_masks, W, b):
    B, L = cycle_curve_data.shape[0], cycle_curve_data.shape[1]
    n_a = B * _K

    xb, sb, se, gf, gl = pl.pallas_call(
        _routing_body,
        grid=(1,),
        in_specs=[
            pl.BlockSpec((B, _E), lambda i: (0, 0)),
            pl.BlockSpec((B, _E), lambda i: (0, 0)),
            pl.BlockSpec((B, L, _C, _S), lambda i: (0, 0, 0, 0)),
        ],
        out_specs=[
            pl.BlockSpec((B * L, _F), lambda i: (0, 0)),
            pl.BlockSpec((n_a, 1), lambda i: (0, 0)),
            pl.BlockSpec((n_a, 1), lambda i: (0, 0)),
            pl.BlockSpec((n_a, 1), lambda i: (0, 0)),
            pl.BlockSpec((1, 1), lambda i: (0, 0)),
        ],
        out_shape=[
            jax.ShapeDtypeStruct((B * L, _F), jnp.bfloat16),
            jax.ShapeDtypeStruct((n_a, 1), jnp.int32),
            jax.ShapeDtypeStruct((n_a, 1), jnp.int32),
            jax.ShapeDtypeStruct((n_a, 1), jnp.float32),
            jax.ShapeDtypeStruct((1, 1), jnp.float32),
        ],
        compiler_params=pltpu.CompilerParams(
            dimension_semantics=("arbitrary",),
        ),
    )(logits, moe_masks, cycle_curve_data)

    xb3 = xb.reshape(B, L, _F)

    out = pl.pallas_call(
        _dispatch_body,
        grid_spec=pltpu.PrefetchScalarGridSpec(
            num_scalar_prefetch=3,
            grid=(n_a,),
            in_specs=[
                pl.BlockSpec((1, L, _F), lambda a, sb, se, gf: (sb[a, 0], 0, 0)),
                pl.BlockSpec((1, _F, _D), lambda a, sb, se, gf: (se[a, 0], 0, 0)),
                pl.BlockSpec((_E, _D), lambda a, sb, se, gf: (0, 0)),
            ],
            out_specs=pl.BlockSpec((B, L, _D), lambda a, sb, se, gf: (0, 0, 0)),
            scratch_shapes=[
                pltpu.VMEM((_F, _D), jnp.bfloat16),
            ],
        ),
        out_shape=jax.ShapeDtypeStruct((B, L, _D), jnp.float32),
        compiler_params=pltpu.CompilerParams(
            dimension_semantics=("arbitrary",),
        ),
    )(sb, se, gf, xb3, W, b)

    return out.astype(jnp.bfloat16), gl[0, 0]


# speedup vs baseline: 1.6841x; 1.6841x over previous
import jax
import jax.numpy as jnp
from jax.experimental import pallas as pl
from jax.experimental.pallas import tpu as pltpu

_E = 8
_K = 2
_D = 768
_C = 3
_S = 512
_F = _C * _S
_EPS = 1e-9


def _moe_body(logits_ref, mask_ref, x_ref, w_ref, b_ref,
              out_ref, gl_ref, gates_ref, acc_ref):
    e = pl.program_id(0)
    n_b = out_ref.shape[0]
    n_l = out_ref.shape[1]

    @pl.when(e == 0)
    def _gating():
        lg = logits_ref[...]
        mk = mask_ref[...]
        m = jnp.where(mk == 1.0, 1.0, 0.0).astype(jnp.float32)
        z = lg - jnp.max(lg, axis=1, keepdims=True)
        ez = jnp.exp(z)
        probs = ez / jnp.sum(ez, axis=1, keepdims=True)
        pm = probs * m
        iota = jax.lax.broadcasted_iota(jnp.int32, pm.shape, 1)
        m1 = jnp.max(pm, axis=1, keepdims=True)
        a1 = jnp.min(jnp.where(pm == m1, iota, _E), axis=1, keepdims=True)
        pm2 = jnp.where(iota == a1, -1.0, pm)
        m2 = jnp.max(pm2, axis=1, keepdims=True)
        a2 = jnp.min(jnp.where(pm2 == m2, iota, _E), axis=1, keepdims=True)
        topk = jnp.logical_or(iota == a1, iota == a2)
        gts = jnp.where(topk, pm, 0.0)
        dn = jnp.sum(gts, axis=1, keepdims=True) + _EPS
        gates_ref[...] = gts / dn
        s = jnp.sum(pm) / jnp.float32(n_b)
        gl_ref[...] = ((1.0 - s) * (1.0 - s)).reshape(1, 1)

    onehot = (jax.lax.broadcasted_iota(jnp.int32, (_E, 1), 0) == e
              ).astype(jnp.float32)
    g_col = jnp.dot(gates_ref[...], onehot)

    y = jnp.dot(x_ref[...], w_ref[0].astype(jnp.bfloat16),
                preferred_element_type=jnp.float32)
    y3 = y.reshape(n_b, n_l, _D) + b_ref[pl.ds(e, 1), :].reshape(1, 1, _D)
    contrib = g_col.reshape(n_b, 1, 1) * y3

    @pl.when(e == 0)
    def _init():
        acc_ref[...] = contrib

    @pl.when(e > 0)
    def _acc():
        acc_ref[...] += contrib

    @pl.when(e == _E - 1)
    def _fin():
        out_ref[...] = acc_ref[...].astype(jnp.bfloat16)


def kernel(cycle_curve_data, logits, moe_masks, W, b):
    B, L = cycle_curve_data.shape[0], cycle_curve_data.shape[1]
    x = cycle_curve_data.reshape(B * L, _F).astype(jnp.bfloat16)

    out, gl = pl.pallas_call(
        _moe_body,
        grid=(_E,),
        in_specs=[
            pl.BlockSpec((B, _E), lambda e: (0, 0)),
            pl.BlockSpec((B, _E), lambda e: (0, 0)),
            pl.BlockSpec((B * L, _F), lambda e: (0, 0)),
            pl.BlockSpec((1, _F, _D), lambda e: (e, 0, 0)),
            pl.BlockSpec((_E, _D), lambda e: (0, 0)),
        ],
        out_specs=[
            pl.BlockSpec((B, L, _D), lambda e: (0, 0, 0)),
            pl.BlockSpec((1, 1), lambda e: (0, 0)),
        ],
        out_shape=[
            jax.ShapeDtypeStruct((B, L, _D), jnp.bfloat16),
            jax.ShapeDtypeStruct((1, 1), jnp.float32),
        ],
        scratch_shapes=[
            pltpu.VMEM((B, _E), jnp.float32),
            pltpu.VMEM((B, L, _D), jnp.float32),
        ],
        compiler_params=pltpu.CompilerParams(
            dimension_semantics=("arbitrary",),
        ),
    )(logits, moe_masks, x, W, b)

    return out, gl[0, 0]
